# Initial kernel scaffold; baseline (speedup 1.0000x reference)
import jax
import jax.numpy as jnp
from jax.experimental import pallas as pl


def kernel(embeddings, targets, W, b, sampled_ids, num_tries):
    raise NotImplementedError("write your pallas kernel here")


# speedup vs baseline: 2.0685x; 2.0685x over previous
import functools
import math

import jax
import jax.numpy as jnp
from jax import lax
from jax.experimental import pallas as pl
from jax.experimental.pallas import tpu as pltpu
from jax.experimental.pallas import tpu_sc as plsc

_NUM_WORDS = 100000
_NUM_SAMPLES = 8192
_EMB_DIM = 128
_BATCH = 4096
_LOG_NW_P1 = math.log(_NUM_WORDS + 1)
_BB = 128
_CHUNK = 128


def _sc_gather(W, b, idx):
    B = idx.shape[0]
    D = W.shape[1]
    info = plsc.get_sparse_core_info()
    nw = info.num_cores * info.num_subcores
    per_w = B // nw
    n_chunks = per_w // _CHUNK
    mesh = plsc.VectorSubcoreMesh(core_axis_name="c", subcore_axis_name="s")

    @functools.partial(
        pl.kernel,
        mesh=mesh,
        out_type=[
            jax.ShapeDtypeStruct((B, D), jnp.float32),
            jax.ShapeDtypeStruct((B,), jnp.float32),
        ],
        scratch_types=[
            pltpu.VMEM((per_w,), jnp.int32),
            pltpu.VMEM((per_w, D), jnp.float32),
            pltpu.VMEM((per_w,), jnp.float32),
            pltpu.SemaphoreType.DMA,
            pltpu.SemaphoreType.DMA,
        ],
    )
    def gather_kernel(w_hbm, b_hbm, idx_hbm, rows_out, bias_out,
                      idx_v, rows_v, bias_v, sem_r, sem_b):
        wid = lax.axis_index("s") * info.num_cores + lax.axis_index("c")
        base = wid * per_w
        pltpu.sync_copy(idx_hbm.at[pl.ds(base, per_w)], idx_v)
        copies = []
        for j in range(n_chunks):
            sl = pl.ds(j * _CHUNK, _CHUNK)
            copies.append(
                pltpu.async_copy(w_hbm.at[idx_v.at[sl]], rows_v.at[sl], sem_r))
            copies.append(
                pltpu.async_copy(b_hbm.at[idx_v.at[sl]], bias_v.at[sl], sem_b))
        for c in copies:
            c.wait()
        pltpu.sync_copy(rows_v, rows_out.at[pl.ds(base, per_w)])
        pltpu.sync_copy(bias_v, bias_out.at[pl.ds(base, per_w)])

    return gather_kernel(W, b, idx)


def _tc_body(nt_ref, emb_ref, sw_ref, tw_ref, sb_ref, tb_ref, tid_ref,
             sid_ref, out_ref):
    i = pl.program_id(0)
    nt = nt_ref[0, 0]
    inv_log = 1.0 / _LOG_NW_P1

    sf = sid_ref[...]
    sp = jnp.log((sf + 2.0) / (sf + 1.0)) * inv_log
    s_exp = 1.0 - jnp.exp(nt * jnp.log1p(-sp))
    s_corr = sb_ref[...] - jnp.log(s_exp + 1e-7)

    tf = tid_ref[...]
    tp = jnp.log((tf + 2.0) / (tf + 1.0)) * inv_log
    t_exp = 1.0 - jnp.exp(nt * jnp.log1p(-tp))

    emb = emb_ref[...]
    tl = (jnp.sum(tw_ref[...] * emb, axis=1, keepdims=True)
          + tb_ref[...] - jnp.log(t_exp + 1e-7))

    logits = lax.dot_general(
        emb.astype(jnp.bfloat16), sw_ref[...],
        (((1,), (1,)), ((), ())),
        preferred_element_type=jnp.float32)
    logits = logits + s_corr
    logits = jnp.where(sf == tf, -10000.0, logits)

    m = jnp.maximum(jnp.max(logits, axis=1, keepdims=True), tl)
    s = jnp.sum(jnp.exp(logits - m), axis=1, keepdims=True) + jnp.exp(tl - m)
    lse = m + jnp.log(s)
    part = jnp.sum(lse - tl)

    @pl.when(i == 0)
    def _():
        out_ref[0, 0] = part

    @pl.when(i != 0)
    def _():
        out_ref[0, 0] += part


def _fused_loss(nt, embeddings, sw, rows, sb, tb, tidf, sidf):
    nb = _BATCH // _BB
    return pl.pallas_call(
        _tc_body,
        grid=(nb,),
        in_specs=[
            pl.BlockSpec(memory_space=pltpu.SMEM),
            pl.BlockSpec((_BB, _EMB_DIM), lambda i: (i, 0)),
            pl.BlockSpec((_NUM_SAMPLES, _EMB_DIM), lambda i: (0, 0)),
            pl.BlockSpec((_BB, _EMB_DIM),
                         lambda i: (_NUM_SAMPLES // _BB + i, 0)),
            pl.BlockSpec((1, _NUM_SAMPLES), lambda i: (0, 0)),
            pl.BlockSpec((_BB, 1), lambda i: (i, 0)),
            pl.BlockSpec((_BB, 1), lambda i: (i, 0)),
            pl.BlockSpec((1, _NUM_SAMPLES), lambda i: (0, 0)),
        ],
        out_specs=pl.BlockSpec(memory_space=pltpu.SMEM),
        out_shape=jax.ShapeDtypeStruct((1, 1), jnp.float32),
        compiler_params=pltpu.CompilerParams(
            dimension_semantics=("arbitrary",)),
    )(nt, embeddings, sw, rows, sb, tb, tidf, sidf)


def kernel(embeddings, targets, W, b, sampled_ids, num_tries):
    idx = jnp.concatenate([sampled_ids, targets]).astype(jnp.int32)
    rows, bias = _sc_gather(W, b, idx)
    sw = rows[:_NUM_SAMPLES].astype(jnp.bfloat16)
    sb = bias[:_NUM_SAMPLES].reshape(1, _NUM_SAMPLES)
    tb = bias[_NUM_SAMPLES:].reshape(_BATCH, 1)
    tidf = targets.astype(jnp.float32).reshape(_BATCH, 1)
    sidf = sampled_ids.astype(jnp.float32).reshape(1, _NUM_SAMPLES)
    nt = jnp.asarray(num_tries, jnp.float32).reshape(1, 1)
    loss = _fused_loss(nt, embeddings, sw, rows, sb, tb, tidf, sidf)
    return loss[0, 0]


# speedup vs baseline: 2.6830x; 1.2971x over previous
import functools
import math

import jax
import jax.numpy as jnp
from jax import lax
from jax.experimental import pallas as pl
from jax.experimental.pallas import tpu as pltpu
from jax.experimental.pallas import tpu_sc as plsc

_NUM_WORDS = 100000
_NUM_SAMPLES = 8192
_EMB_DIM = 128
_BATCH = 4096
_LOG_NW_P1 = math.log(_NUM_WORDS + 1)
_BB = 128
_CHUNK = 128


def _sc_gather(W, b, idx):
    B = idx.shape[0]
    D = W.shape[1]
    info = plsc.get_sparse_core_info()
    nw = info.num_cores * info.num_subcores
    per_w = B // nw
    n_chunks = per_w // _CHUNK
    mesh = plsc.VectorSubcoreMesh(core_axis_name="c", subcore_axis_name="s")

    @functools.partial(
        pl.kernel,
        mesh=mesh,
        out_type=[
            jax.ShapeDtypeStruct((B, D), jnp.float32),
            jax.ShapeDtypeStruct((B,), jnp.float32),
        ],
        scratch_types=[
            pltpu.VMEM((per_w,), jnp.int32),
            pltpu.VMEM((per_w, D), jnp.float32),
            pltpu.VMEM((per_w,), jnp.float32),
            pltpu.SemaphoreType.DMA,
            pltpu.SemaphoreType.DMA,
        ],
    )
    def gather_kernel(w_hbm, b_hbm, idx_hbm, rows_out, bias_out,
                      idx_v, rows_v, bias_v, sem_r, sem_b):
        wid = lax.axis_index("s") * info.num_cores + lax.axis_index("c")
        base = wid * per_w
        pltpu.sync_copy(idx_hbm.at[pl.ds(base, per_w)], idx_v)
        copies = []
        for j in range(n_chunks):
            sl = pl.ds(j * _CHUNK, _CHUNK)
            copies.append(
                pltpu.async_copy(w_hbm.at[idx_v.at[sl]], rows_v.at[sl], sem_r))
            copies.append(
                pltpu.async_copy(b_hbm.at[idx_v.at[sl]], bias_v.at[sl], sem_b))
        for c in copies:
            c.wait()
        pltpu.sync_copy(rows_v, rows_out.at[pl.ds(base, per_w)])
        pltpu.sync_copy(bias_v, bias_out.at[pl.ds(base, per_w)])

    return gather_kernel(W, b, idx)


_SHIFT = 16.0
_LOG2E = 1.4426950408889634


def _tc_body(nt_ref, emb_ref, embs_ref, sw_ref, tw_ref, sb_ref, tb_ref,
             tid_ref, sid_ref, out_ref, corr_ref):
    i = pl.program_id(0)
    nt = nt_ref[0, 0]
    inv_log = 1.0 / _LOG_NW_P1

    @pl.when(i == 0)
    def _():
        sf0 = sid_ref[...]
        sp = jnp.log((sf0 + 2.0) / (sf0 + 1.0)) * inv_log
        s_exp = 1.0 - jnp.exp(nt * jnp.log1p(-sp))
        corr_ref[...] = (sb_ref[...] - jnp.log(s_exp + 1e-7)
                         - _SHIFT) * _LOG2E

    tf = tid_ref[...]
    tp = jnp.log((tf + 2.0) / (tf + 1.0)) * inv_log
    t_exp = 1.0 - jnp.exp(nt * jnp.log1p(-tp))

    emb = emb_ref[...]
    tl = (jnp.sum(tw_ref[...] * emb, axis=1, keepdims=True)
          + tb_ref[...] - jnp.log(t_exp + 1e-7))

    y = lax.dot_general(
        embs_ref[...], sw_ref[...],
        (((1,), (1,)), ((), ())),
        preferred_element_type=jnp.float32)
    y = y + corr_ref[...]
    y = jnp.where(sid_ref[...] == tf, -20000.0, y)

    s = (jnp.sum(jnp.exp2(y), axis=1, keepdims=True)
         + jnp.exp2((tl - _SHIFT) * _LOG2E))
    lse = _SHIFT + jnp.log(s)
    part = jnp.sum(lse - tl)

    @pl.when(i == 0)
    def _():
        out_ref[0, 0] = part

    @pl.when(i != 0)
    def _():
        out_ref[0, 0] += part


def _fused_loss(nt, embeddings, embs, sw, rows, sb, tb, tidf, sidf):
    nb = _BATCH // _BB
    return pl.pallas_call(
        _tc_body,
        grid=(nb,),
        in_specs=[
            pl.BlockSpec(memory_space=pltpu.SMEM),
            pl.BlockSpec((_BB, _EMB_DIM), lambda i: (i, 0)),
            pl.BlockSpec((_BB, _EMB_DIM), lambda i: (i, 0)),
            pl.BlockSpec((_NUM_SAMPLES, _EMB_DIM), lambda i: (0, 0)),
            pl.BlockSpec((_BB, _EMB_DIM),
                         lambda i: (_NUM_SAMPLES // _BB + i, 0)),
            pl.BlockSpec((1, _NUM_SAMPLES), lambda i: (0, 0)),
            pl.BlockSpec((_BB, 1), lambda i: (i, 0)),
            pl.BlockSpec((_BB, 1), lambda i: (i, 0)),
            pl.BlockSpec((1, _NUM_SAMPLES), lambda i: (0, 0)),
        ],
        out_specs=pl.BlockSpec(memory_space=pltpu.SMEM),
        out_shape=jax.ShapeDtypeStruct((1, 1), jnp.float32),
        scratch_shapes=[pltpu.VMEM((1, _NUM_SAMPLES), jnp.float32)],
        compiler_params=pltpu.CompilerParams(
            dimension_semantics=("arbitrary",)),
    )(nt, embeddings, embs, sw, rows, sb, tb, tidf, sidf)


def kernel(embeddings, targets, W, b, sampled_ids, num_tries):
    idx = jnp.concatenate([sampled_ids, targets]).astype(jnp.int32)
    rows, bias = _sc_gather(W, b, idx)
    sw = rows[:_NUM_SAMPLES].astype(jnp.bfloat16)
    embs = (embeddings * _LOG2E).astype(jnp.bfloat16)
    sb = bias[:_NUM_SAMPLES].reshape(1, _NUM_SAMPLES)
    tb = bias[_NUM_SAMPLES:].reshape(_BATCH, 1)
    tidf = targets.astype(jnp.float32).reshape(_BATCH, 1)
    sidf = sampled_ids.astype(jnp.float32).reshape(1, _NUM_SAMPLES)
    nt = jnp.asarray(num_tries, jnp.float32).reshape(1, 1)
    loss = _fused_loss(nt, embeddings, embs, sw, rows, sb, tb, tidf, sidf)
    return loss[0, 0]


# speedup vs baseline: 2.8596x; 1.0658x over previous
import functools
import math

import jax
import jax.numpy as jnp
from jax import lax
from jax.experimental import pallas as pl
from jax.experimental.pallas import tpu as pltpu
from jax.experimental.pallas import tpu_sc as plsc

_NUM_WORDS = 100000
_NUM_SAMPLES = 8192
_EMB_DIM = 128
_BATCH = 4096
_LOG_NW_P1 = math.log(_NUM_WORDS + 1)
_BB = 128
_CHUNK = 128
_SHIFT = 16.0
_LOG2E = 1.4426950408889634


def _sc_gather(W, b, sampled_ids, targets):
    D = W.shape[1]
    info = plsc.get_sparse_core_info()
    nw = info.num_cores * info.num_subcores
    s_per = _NUM_SAMPLES // nw
    t_per = _BATCH // nw
    mesh = plsc.VectorSubcoreMesh(core_axis_name="c", subcore_axis_name="s")

    @functools.partial(
        pl.kernel,
        mesh=mesh,
        out_type=[
            jax.ShapeDtypeStruct((_NUM_SAMPLES + _BATCH, D), jnp.float32),
            jax.ShapeDtypeStruct((_NUM_SAMPLES,), jnp.float32),
            jax.ShapeDtypeStruct((_BATCH,), jnp.float32),
        ],
        scratch_types=[
            pltpu.VMEM((s_per,), jnp.int32),
            pltpu.VMEM((t_per,), jnp.int32),
            pltpu.VMEM((s_per, D), jnp.float32),
            pltpu.VMEM((t_per, D), jnp.float32),
            pltpu.VMEM((s_per,), jnp.float32),
            pltpu.VMEM((t_per,), jnp.float32),
            pltpu.SemaphoreType.DMA,
            pltpu.SemaphoreType.DMA,
        ],
    )
    def gather_kernel(w_hbm, b_hbm, sid_hbm, tid_hbm,
                      rows_out, sbias_out, tbias_out,
                      sidx_v, tidx_v, srows_v, trows_v, sbias_v, tbias_v,
                      sem_r, sem_b):
        wid = lax.axis_index("s") * info.num_cores + lax.axis_index("c")
        s_base = wid * s_per
        t_base = wid * t_per
        pltpu.sync_copy(sid_hbm.at[pl.ds(s_base, s_per)], sidx_v)
        pltpu.sync_copy(tid_hbm.at[pl.ds(t_base, t_per)], tidx_v)
        copies = []
        for j in range(s_per // _CHUNK):
            sl = pl.ds(j * _CHUNK, _CHUNK)
            copies.append(
                pltpu.async_copy(w_hbm.at[sidx_v.at[sl]], srows_v.at[sl],
                                 sem_r))
            copies.append(
                pltpu.async_copy(b_hbm.at[sidx_v.at[sl]], sbias_v.at[sl],
                                 sem_b))
        copies.append(pltpu.async_copy(w_hbm.at[tidx_v], trows_v, sem_r))
        copies.append(pltpu.async_copy(b_hbm.at[tidx_v], tbias_v, sem_b))
        for c in copies:
            c.wait()
        pltpu.sync_copy(srows_v, rows_out.at[pl.ds(s_base, s_per)])
        pltpu.sync_copy(trows_v,
                        rows_out.at[pl.ds(_NUM_SAMPLES + t_base, t_per)])
        pltpu.sync_copy(sbias_v, sbias_out.at[pl.ds(s_base, s_per)])
        pltpu.sync_copy(tbias_v, tbias_out.at[pl.ds(t_base, t_per)])

    return gather_kernel(W, b, sampled_ids, targets)


def _tc_body(nt_ref, emb_ref, sw_ref, tw_ref, sb_ref, tb_ref,
             tid_ref, sid_ref, out_ref, corr_ref, swb_ref):
    i = pl.program_id(0)
    nt = nt_ref[0, 0]
    inv_log = 1.0 / _LOG_NW_P1

    @pl.when(i == 0)
    def _():
        swb_ref[...] = sw_ref[...].astype(jnp.bfloat16)
        sf0 = sid_ref[...].astype(jnp.float32)
        sp = jnp.log((sf0 + 2.0) / (sf0 + 1.0)) * inv_log
        s_exp = 1.0 - jnp.exp(nt * jnp.log1p(-sp))
        corr_ref[...] = (sb_ref[...] - jnp.log(s_exp + 1e-7)
                         - _SHIFT) * _LOG2E

    emb = emb_ref[...]
    embs = (emb * _LOG2E).astype(jnp.bfloat16)
    y = lax.dot_general(
        embs, swb_ref[...],
        (((1,), (1,)), ((), ())),
        preferred_element_type=jnp.float32)
    y = y + corr_ref[...]

    tid_row = tid_ref[...]
    tid_col = jnp.transpose(tid_row)
    y = jnp.where(sid_ref[...] == tid_col, -20000.0, y)

    s_col = jnp.sum(jnp.exp2(y), axis=1, keepdims=True)
    s_row = jnp.transpose(s_col)

    tdot_col = jnp.sum(tw_ref[...] * emb, axis=1, keepdims=True)
    tdot_row = jnp.transpose(tdot_col)

    tf = tid_row.astype(jnp.float32)
    tp = jnp.log((tf + 2.0) / (tf + 1.0)) * inv_log
    t_exp = 1.0 - jnp.exp(nt * jnp.log1p(-tp))
    tl = tdot_row + tb_ref[...] - jnp.log(t_exp + 1e-7)

    s_tot = s_row + jnp.exp2((tl - _SHIFT) * _LOG2E)
    part = jnp.sum(_SHIFT + jnp.log(s_tot) - tl)

    @pl.when(i == 0)
    def _():
        out_ref[0, 0] = part

    @pl.when(i != 0)
    def _():
        out_ref[0, 0] += part


def _fused_loss(nt, embeddings, rows, sbias, tbias, tid2, sid2):
    nb = _BATCH // _BB
    return pl.pallas_call(
        _tc_body,
        grid=(nb,),
        in_specs=[
            pl.BlockSpec(memory_space=pltpu.SMEM),
            pl.BlockSpec((_BB, _EMB_DIM), lambda i: (i, 0)),
            pl.BlockSpec((_NUM_SAMPLES, _EMB_DIM), lambda i: (0, 0)),
            pl.BlockSpec((_BB, _EMB_DIM),
                         lambda i: (_NUM_SAMPLES // _BB + i, 0)),
            pl.BlockSpec((1, _NUM_SAMPLES), lambda i: (0, 0)),
            pl.BlockSpec((1, _BB), lambda i: (0, i)),
            pl.BlockSpec((1, _BB), lambda i: (0, i)),
            pl.BlockSpec((1, _NUM_SAMPLES), lambda i: (0, 0)),
        ],
        out_specs=pl.BlockSpec(memory_space=pltpu.SMEM),
        out_shape=jax.ShapeDtypeStruct((1, 1), jnp.float32),
        scratch_shapes=[
            pltpu.VMEM((1, _NUM_SAMPLES), jnp.float32),
            pltpu.VMEM((_NUM_SAMPLES, _EMB_DIM), jnp.bfloat16),
        ],
        compiler_params=pltpu.CompilerParams(
            dimension_semantics=("arbitrary",)),
    )(nt, embeddings, rows, rows, sbias, tbias, tid2, sid2)


def kernel(embeddings, targets, W, b, sampled_ids, num_tries):
    rows, sbias, tbias = _sc_gather(W, b, sampled_ids, targets)
    sb = sbias.reshape(1, _NUM_SAMPLES)
    tb = tbias.reshape(1, _BATCH)
    tid2 = targets.reshape(1, _BATCH)
    sid2 = sampled_ids.reshape(1, _NUM_SAMPLES)
    nt = jnp.asarray(num_tries, jnp.float32).reshape(1, 1)
    loss = _fused_loss(nt, embeddings, rows, sb, tb, tid2, sid2)
    return loss[0, 0]


# speedup vs baseline: 3.0996x; 1.0839x over previous
import functools
import math

import jax
import jax.numpy as jnp
from jax import lax
from jax.experimental import pallas as pl
from jax.experimental.pallas import tpu as pltpu
from jax.experimental.pallas import tpu_sc as plsc

_NUM_WORDS = 100000
_NUM_SAMPLES = 8192
_EMB_DIM = 128
_BATCH = 4096
_LOG_NW_P1 = math.log(_NUM_WORDS + 1)
_BB = 256
_CHUNK = 128
_SHIFT = 16.0
_LOG2E = 1.4426950408889634


def _sc_gather(W, b, sampled_ids, targets):
    D = W.shape[1]
    info = plsc.get_sparse_core_info()
    nw = info.num_cores * info.num_subcores
    s_per = _NUM_SAMPLES // nw
    t_per = _BATCH // nw
    mesh = plsc.VectorSubcoreMesh(core_axis_name="c", subcore_axis_name="s")

    @functools.partial(
        pl.kernel,
        mesh=mesh,
        out_type=[
            jax.ShapeDtypeStruct((_NUM_SAMPLES + _BATCH, D), jnp.float32),
            jax.ShapeDtypeStruct((_NUM_SAMPLES,), jnp.float32),
            jax.ShapeDtypeStruct((_BATCH,), jnp.float32),
        ],
        scratch_types=[
            pltpu.VMEM((s_per,), jnp.int32),
            pltpu.VMEM((t_per,), jnp.int32),
            pltpu.VMEM((s_per, D), jnp.float32),
            pltpu.VMEM((t_per, D), jnp.float32),
            pltpu.VMEM((s_per,), jnp.float32),
            pltpu.VMEM((t_per,), jnp.float32),
            pltpu.SemaphoreType.DMA,
            pltpu.SemaphoreType.DMA,
        ],
    )
    def gather_kernel(w_hbm, b_hbm, sid_hbm, tid_hbm,
                      rows_out, sbias_out, tbias_out,
                      sidx_v, tidx_v, srows_v, trows_v, sbias_v, tbias_v,
                      sem_r, sem_b):
        wid = lax.axis_index("s") * info.num_cores + lax.axis_index("c")
        s_base = wid * s_per
        t_base = wid * t_per
        pltpu.sync_copy(sid_hbm.at[pl.ds(s_base, s_per)], sidx_v)
        pltpu.sync_copy(tid_hbm.at[pl.ds(t_base, t_per)], tidx_v)
        copies = []
        for j in range(s_per // _CHUNK):
            sl = pl.ds(j * _CHUNK, _CHUNK)
            copies.append(
                pltpu.async_copy(w_hbm.at[sidx_v.at[sl]], srows_v.at[sl],
                                 sem_r))
            copies.append(
                pltpu.async_copy(b_hbm.at[sidx_v.at[sl]], sbias_v.at[sl],
                                 sem_b))
        copies.append(pltpu.async_copy(w_hbm.at[tidx_v], trows_v, sem_r))
        copies.append(pltpu.async_copy(b_hbm.at[tidx_v], tbias_v, sem_b))
        for c in copies:
            c.wait()
        pltpu.sync_copy(srows_v, rows_out.at[pl.ds(s_base, s_per)])
        pltpu.sync_copy(trows_v,
                        rows_out.at[pl.ds(_NUM_SAMPLES + t_base, t_per)])
        pltpu.sync_copy(sbias_v, sbias_out.at[pl.ds(s_base, s_per)])
        pltpu.sync_copy(tbias_v, tbias_out.at[pl.ds(t_base, t_per)])

    return gather_kernel(W, b, sampled_ids, targets)


def _tc_body(nt_ref, emb_ref, sw_ref, tw_ref, sb_ref, tb_ref,
             tid_ref, sid_ref, out_ref, corr_ref, swb_ref):
    i = pl.program_id(0)
    nt = nt_ref[0, 0]
    inv_log = 1.0 / _LOG_NW_P1

    @pl.when(i == 0)
    def _():
        swb_ref[...] = sw_ref[...].astype(jnp.bfloat16)
        sf0 = sid_ref[...].astype(jnp.float32)
        sp = jnp.log((sf0 + 2.0) / (sf0 + 1.0)) * inv_log
        s_exp = 1.0 - jnp.exp(nt * jnp.log1p(-sp))
        corr_ref[...] = (sb_ref[...] - jnp.log(s_exp + 1e-7)
                         - _SHIFT) * _LOG2E

    emb = emb_ref[...]
    embs = (emb * _LOG2E).astype(jnp.bfloat16)
    y = lax.dot_general(
        embs, swb_ref[...],
        (((1,), (1,)), ((), ())),
        preferred_element_type=jnp.float32)
    y = y + corr_ref[...]

    tid_row = tid_ref[...]
    tid_col = jnp.transpose(tid_row)
    y = jnp.where(sid_ref[...] == tid_col, -20000.0, y)

    s_col = jnp.sum(jnp.exp2(y), axis=1, keepdims=True)
    s_row = jnp.transpose(s_col)

    tdot_col = jnp.sum(tw_ref[...] * emb, axis=1, keepdims=True)
    tdot_row = jnp.transpose(tdot_col)

    tf = tid_row.astype(jnp.float32)
    tp = jnp.log((tf + 2.0) / (tf + 1.0)) * inv_log
    t_exp = 1.0 - jnp.exp(nt * jnp.log1p(-tp))
    tl = tdot_row + tb_ref[...] - jnp.log(t_exp + 1e-7)

    s_tot = s_row + jnp.exp2((tl - _SHIFT) * _LOG2E)
    part = jnp.sum(_SHIFT + jnp.log(s_tot) - tl)

    @pl.when(i == 0)
    def _():
        out_ref[0, 0] = part

    @pl.when(i != 0)
    def _():
        out_ref[0, 0] += part


def _fused_loss(nt, embeddings, rows, sbias, tbias, tid2, sid2):
    nb = _BATCH // _BB
    return pl.pallas_call(
        _tc_body,
        grid=(nb,),
        in_specs=[
            pl.BlockSpec(memory_space=pltpu.SMEM),
            pl.BlockSpec((_BB, _EMB_DIM), lambda i: (i, 0)),
            pl.BlockSpec((_NUM_SAMPLES, _EMB_DIM), lambda i: (0, 0)),
            pl.BlockSpec((_BB, _EMB_DIM),
                         lambda i: (_NUM_SAMPLES // _BB + i, 0)),
            pl.BlockSpec((1, _NUM_SAMPLES), lambda i: (0, 0)),
            pl.BlockSpec((1, _BB), lambda i: (0, i)),
            pl.BlockSpec((1, _BB), lambda i: (0, i)),
            pl.BlockSpec((1, _NUM_SAMPLES), lambda i: (0, 0)),
        ],
        out_specs=pl.BlockSpec(memory_space=pltpu.SMEM),
        out_shape=jax.ShapeDtypeStruct((1, 1), jnp.float32),
        scratch_shapes=[
            pltpu.VMEM((1, _NUM_SAMPLES), jnp.float32),
            pltpu.VMEM((_NUM_SAMPLES, _EMB_DIM), jnp.bfloat16),
        ],
        compiler_params=pltpu.CompilerParams(
            dimension_semantics=("arbitrary",)),
    )(nt, embeddings, rows, rows, sbias, tbias, tid2, sid2)


def kernel(embeddings, targets, W, b, sampled_ids, num_tries):
    rows, sbias, tbias = _sc_gather(W, b, sampled_ids, targets)
    sb = sbias.reshape(1, _NUM_SAMPLES)
    tb = tbias.reshape(1, _BATCH)
    tid2 = targets.reshape(1, _BATCH)
    sid2 = sampled_ids.reshape(1, _NUM_SAMPLES)
    nt = jnp.asarray(num_tries, jnp.float32).reshape(1, 1)
    loss = _fused_loss(nt, embeddings, rows, sb, tb, tid2, sid2)
    return loss[0, 0]


# speedup vs baseline: 3.3094x; 1.0677x over previous
import functools
import math

import jax
import jax.numpy as jnp
from jax import lax
from jax.experimental import pallas as pl
from jax.experimental.pallas import tpu as pltpu
from jax.experimental.pallas import tpu_sc as plsc

_NUM_WORDS = 100000
_NUM_SAMPLES = 8192
_EMB_DIM = 128
_BATCH = 4096
_LOG_NW_P1 = math.log(_NUM_WORDS + 1)
_BB = 512
_CHUNK = 128
_SHIFT = 16.0
_LOG2E = 1.4426950408889634


def _sc_gather(W, b, sampled_ids, targets):
    D = W.shape[1]
    info = plsc.get_sparse_core_info()
    nw = info.num_cores * info.num_subcores
    s_per = _NUM_SAMPLES // nw
    t_per = _BATCH // nw
    mesh = plsc.VectorSubcoreMesh(core_axis_name="c", subcore_axis_name="s")

    @functools.partial(
        pl.kernel,
        mesh=mesh,
        out_type=[
            jax.ShapeDtypeStruct((_NUM_SAMPLES + _BATCH, D), jnp.float32),
            jax.ShapeDtypeStruct((_NUM_SAMPLES,), jnp.float32),
            jax.ShapeDtypeStruct((_BATCH,), jnp.float32),
        ],
        scratch_types=[
            pltpu.VMEM((s_per,), jnp.int32),
            pltpu.VMEM((t_per,), jnp.int32),
            pltpu.VMEM((s_per, D), jnp.float32),
            pltpu.VMEM((t_per, D), jnp.float32),
            pltpu.VMEM((s_per,), jnp.float32),
            pltpu.VMEM((t_per,), jnp.float32),
            pltpu.SemaphoreType.DMA,
            pltpu.SemaphoreType.DMA,
        ],
    )
    def gather_kernel(w_hbm, b_hbm, sid_hbm, tid_hbm,
                      rows_out, sbias_out, tbias_out,
                      sidx_v, tidx_v, srows_v, trows_v, sbias_v, tbias_v,
                      sem_r, sem_b):
        wid = lax.axis_index("s") * info.num_cores + lax.axis_index("c")
        s_base = wid * s_per
        t_base = wid * t_per
        pltpu.sync_copy(sid_hbm.at[pl.ds(s_base, s_per)], sidx_v)
        pltpu.sync_copy(tid_hbm.at[pl.ds(t_base, t_per)], tidx_v)
        copies = []
        for j in range(s_per // _CHUNK):
            sl = pl.ds(j * _CHUNK, _CHUNK)
            copies.append(
                pltpu.async_copy(w_hbm.at[sidx_v.at[sl]], srows_v.at[sl],
                                 sem_r))
            copies.append(
                pltpu.async_copy(b_hbm.at[sidx_v.at[sl]], sbias_v.at[sl],
                                 sem_b))
        copies.append(pltpu.async_copy(w_hbm.at[tidx_v], trows_v, sem_r))
        copies.append(pltpu.async_copy(b_hbm.at[tidx_v], tbias_v, sem_b))
        for c in copies:
            c.wait()
        pltpu.sync_copy(srows_v, rows_out.at[pl.ds(s_base, s_per)])
        pltpu.sync_copy(trows_v,
                        rows_out.at[pl.ds(_NUM_SAMPLES + t_base, t_per)])
        pltpu.sync_copy(sbias_v, sbias_out.at[pl.ds(s_base, s_per)])
        pltpu.sync_copy(tbias_v, tbias_out.at[pl.ds(t_base, t_per)])

    return gather_kernel(W, b, sampled_ids, targets)


def _tc_body(nt_ref, emb_ref, sw_ref, tw_ref, sb_ref, tb_ref,
             tid_ref, sid_ref, out_ref, corr_ref, swb_ref):
    i = pl.program_id(0)
    nt = nt_ref[0, 0]
    inv_log = 1.0 / _LOG_NW_P1

    @pl.when(i == 0)
    def _():
        swb_ref[...] = sw_ref[...].astype(jnp.bfloat16)
        sf0 = sid_ref[...].astype(jnp.float32)
        sp = jnp.log((sf0 + 2.0) / (sf0 + 1.0)) * inv_log
        s_exp = 1.0 - jnp.exp(nt * jnp.log1p(-sp))
        corr_ref[...] = (sb_ref[...] - jnp.log(s_exp + 1e-7)
                         - _SHIFT) * _LOG2E

    emb = emb_ref[...]
    embs = (emb * _LOG2E).astype(jnp.bfloat16)
    y = lax.dot_general(
        embs, swb_ref[...],
        (((1,), (1,)), ((), ())),
        preferred_element_type=jnp.float32)
    y = y + corr_ref[...]

    tid_row = tid_ref[...]
    tid_col = jnp.transpose(tid_row)
    y = jnp.where(sid_ref[...] == tid_col, -20000.0, y)

    s_col = jnp.sum(jnp.exp2(y), axis=1, keepdims=True)
    s_row = jnp.transpose(s_col)

    tdot_col = jnp.sum(tw_ref[...] * emb, axis=1, keepdims=True)
    tdot_row = jnp.transpose(tdot_col)

    tf = tid_row.astype(jnp.float32)
    tp = jnp.log((tf + 2.0) / (tf + 1.0)) * inv_log
    t_exp = 1.0 - jnp.exp(nt * jnp.log1p(-tp))
    tl = tdot_row + tb_ref[...] - jnp.log(t_exp + 1e-7)

    s_tot = s_row + jnp.exp2((tl - _SHIFT) * _LOG2E)
    part = jnp.sum(_SHIFT + jnp.log(s_tot) - tl)

    @pl.when(i == 0)
    def _():
        out_ref[0, 0] = part

    @pl.when(i != 0)
    def _():
        out_ref[0, 0] += part


def _fused_loss(nt, embeddings, rows, sbias, tbias, tid2, sid2):
    nb = _BATCH // _BB
    return pl.pallas_call(
        _tc_body,
        grid=(nb,),
        in_specs=[
            pl.BlockSpec(memory_space=pltpu.SMEM),
            pl.BlockSpec((_BB, _EMB_DIM), lambda i: (i, 0)),
            pl.BlockSpec((_NUM_SAMPLES, _EMB_DIM), lambda i: (0, 0)),
            pl.BlockSpec((_BB, _EMB_DIM),
                         lambda i: (_NUM_SAMPLES // _BB + i, 0)),
            pl.BlockSpec((1, _NUM_SAMPLES), lambda i: (0, 0)),
            pl.BlockSpec((1, _BB), lambda i: (0, i)),
            pl.BlockSpec((1, _BB), lambda i: (0, i)),
            pl.BlockSpec((1, _NUM_SAMPLES), lambda i: (0, 0)),
        ],
        out_specs=pl.BlockSpec(memory_space=pltpu.SMEM),
        out_shape=jax.ShapeDtypeStruct((1, 1), jnp.float32),
        scratch_shapes=[
            pltpu.VMEM((1, _NUM_SAMPLES), jnp.float32),
            pltpu.VMEM((_NUM_SAMPLES, _EMB_DIM), jnp.bfloat16),
        ],
        compiler_params=pltpu.CompilerParams(
            dimension_semantics=("arbitrary",)),
    )(nt, embeddings, rows, rows, sbias, tbias, tid2, sid2)


def kernel(embeddings, targets, W, b, sampled_ids, num_tries):
    rows, sbias, tbias = _sc_gather(W, b, sampled_ids, targets)
    sb = sbias.reshape(1, _NUM_SAMPLES)
    tb = tbias.reshape(1, _BATCH)
    tid2 = targets.reshape(1, _BATCH)
    sid2 = sampled_ids.reshape(1, _NUM_SAMPLES)
    nt = jnp.asarray(num_tries, jnp.float32).reshape(1, 1)
    loss = _fused_loss(nt, embeddings, rows, sb, tb, tid2, sid2)
    return loss[0, 0]
